# Initial kernel scaffold; baseline (speedup 1.0000x reference)
import jax
import jax.numpy as jnp
from jax.experimental import pallas as pl


def kernel(local_entity, q2e_adj_mat, kb_fact_rel, query_text, answer_dist, fact_head, fact_tail, word_emb_table, entity_kge_table, rel_emb_table, W_ent, b_ent, W_rel, b_rel, Wi_f, Wh_f, b_f, Wi_b, Wh_b, b_b, W_lin, b_lin, W_trans, b_trans, W_score, b_score):
    raise NotImplementedError("write your pallas kernel here")


# speedup vs baseline: 3.4281x; 3.4281x over previous
import functools

import jax
import jax.numpy as jnp
from jax import lax
from jax.experimental import pallas as pl
from jax.experimental.pallas import tpu as pltpu
from jax.experimental.pallas import tpu_sc as plsc

B, M, F, Q = 8, 5000, 20000, 16
NE, NR, NW = 100000, 200, 40000
H, WD, KD, L = 128, 128, 128, 3
VERY_NEG = -100000000000.0

NC, NS = 2, 16
NWK = NC * NS
M_PAD = 5120
F_PAD = 20480
NRP = 256
CHK = 128
ROWS_PT = F_PAD // NS
STRIPE = M_PAD // NS
BPC = B // NC

_mesh = functools.partial(
    plsc.VectorSubcoreMesh, core_axis_name="c", subcore_axis_name="s")


def _wid():
    return lax.axis_index("s") * NC + lax.axis_index("c")


def _make_gather(n_per_w, k):
    nchunks = n_per_w // k

    @functools.partial(
        pl.kernel,
        out_type=jax.ShapeDtypeStruct((NWK * n_per_w, H), jnp.float32),
        mesh=_mesh(),
        compiler_params=pltpu.CompilerParams(needs_layout_passes=False),
        scratch_types=[
            pltpu.VMEM((k,), jnp.int32),
            pltpu.VMEM((k, H), jnp.float32),
            pltpu.SemaphoreType.DMA,
        ],
    )
    def gather_kernel(tab_hbm, idx_hbm, out_hbm, idx_v, rows_v, sem):
        w = _wid()

        def body(j, c):
            base = w * n_per_w + j * k
            pltpu.sync_copy(idx_hbm.at[pl.ds(base, k)], idx_v)
            pltpu.async_copy(tab_hbm.at[idx_v], rows_v, sem).wait()
            pltpu.sync_copy(rows_v, out_hbm.at[pl.ds(base, k)])
            return c

        lax.fori_loop(0, nchunks, body, 0)

    return gather_kernel


_gather_ent = _make_gather(M_PAD * B // NWK, CHK)
_gather_word = _make_gather(256 // NWK, 256 // NWK)


@functools.partial(
    pl.kernel,
    out_type=jax.ShapeDtypeStruct((B * M_PAD, H), jnp.float32),
    mesh=_mesh(),
    compiler_params=pltpu.CompilerParams(needs_layout_passes=False),
    scratch_types=[
        pltpu.VMEM((CHK,), jnp.int32),
        pltpu.VMEM((CHK,), jnp.int32),
        pltpu.VMEM((CHK,), jnp.int32),
        pltpu.VMEM((CHK, H), jnp.float32),
        pltpu.VMEM((CHK, H), jnp.float32),
        pltpu.VMEM((CHK, H), jnp.float32),
        pltpu.VMEM((NRP,), jnp.float32),
        pltpu.VMEM_SHARED((M_PAD, H), jnp.float32),
        pltpu.SemaphoreType.DMA,
        pltpu.SemaphoreType.DMA,
    ],
)
def _facts(emb_hbm, heads_hbm, tails_hbm, rels_hbm, reltab_hbm, scoretab_hbm,
           zeros_hbm, out_hbm, hidx, tidx, ridx, hrow, rrow, orow, score_v,
           agg_sh, sem1, sem2):
    cid = lax.axis_index("c")
    sid = lax.axis_index("s")
    iot = lax.iota(jnp.int32, 16)

    for bi in range(BPC):
        b = cid * BPC + bi
        pltpu.sync_copy(zeros_hbm, agg_sh.at[pl.ds(sid * STRIPE, STRIPE)])
        pltpu.sync_copy(scoretab_hbm.at[b], score_v)
        plsc.subcore_barrier()

        def chunk(j, c, b=b):
            fbase = sid * ROWS_PT + j * CHK
            pltpu.sync_copy(heads_hbm.at[b, pl.ds(fbase, CHK)], hidx)
            pltpu.sync_copy(tails_hbm.at[b, pl.ds(fbase, CHK)], tidx)
            pltpu.sync_copy(rels_hbm.at[b, pl.ds(fbase, CHK)], ridx)
            g1 = pltpu.async_copy(emb_hbm.at[hidx], hrow, sem1)
            g2 = pltpu.async_copy(reltab_hbm.at[ridx], rrow, sem2)
            g1.wait()
            g2.wait()

            def fact(k2, c2):
                k16 = jnp.zeros((16,), jnp.int32) + k2
                ri = plsc.load_gather(ridx, [k16])
                sk = plsc.load_gather(score_v, [ri])
                for cc in range(H // 16):
                    cols = iot + (cc * 16)
                    hv = plsc.load_gather(hrow, [k16, cols])
                    rv = plsc.load_gather(rrow, [k16, cols])
                    plsc.store_scatter(
                        orow, [k16, cols], jnp.maximum(hv + rv, 0.0) * sk)
                return c2

            lax.fori_loop(0, CHK, fact, 0)
            pltpu.sync_copy(orow, agg_sh.at[tidx], add=True)
            return c

        lax.fori_loop(0, ROWS_PT // CHK, chunk, 0)
        plsc.subcore_barrier()
        pltpu.sync_copy(
            agg_sh.at[pl.ds(sid * STRIPE, STRIPE)],
            out_hbm.at[pl.ds(b * M_PAD + sid * STRIPE, STRIPE)])
        plsc.subcore_barrier()


def _sigmoid(x):
    return 1.0 / (1.0 + jnp.exp(-x))


def _encode_body(qw, wif, whf, bf, wib, whb, bb, relp, wrel, brel,
                 qne_o, st_o, rel_o):
    xf = jnp.dot(qw[...], wif[...], preferred_element_type=jnp.float32) + bf[...]
    xb = jnp.dot(qw[...], wib[...], preferred_element_type=jnp.float32) + bb[...]

    def scan_dir(x, wh, reverse):
        h = jnp.zeros((B, H), jnp.float32)
        c = jnp.zeros((B, H), jnp.float32)
        for t in range(Q):
            tt = (Q - 1 - t) if reverse else t
            z = x[tt * B:(tt + 1) * B, :] + jnp.dot(
                h, wh[...], preferred_element_type=jnp.float32)
            i = _sigmoid(z[:, 0:H])
            f = _sigmoid(z[:, H:2 * H])
            g = jnp.tanh(z[:, 2 * H:3 * H])
            o = _sigmoid(z[:, 3 * H:4 * H])
            c = f * c + i * g
            h = o * jnp.tanh(c)
        return h

    hf = scan_dir(xf, whf, False)
    hb = scan_dir(xb, whb, True)
    qne = (hf + hb) * 0.5
    rel = jnp.dot(relp[...], wrel[...], preferred_element_type=jnp.float32) + brel[...]
    score = lax.dot_general(qne, rel, (((1,), (1,)), ((), ())),
                            preferred_element_type=jnp.float32)
    col = lax.broadcasted_iota(jnp.int32, (B, NRP), 1)
    st = _sigmoid(score) * (col != NR).astype(jnp.float32)
    qne_o[...] = qne
    st_o[...] = st
    rel_o[...] = rel


def _encode(qw, wif, whf, bf, wib, whb, bb, relp, wrel, brel):
    return pl.pallas_call(
        _encode_body,
        out_shape=[
            jax.ShapeDtypeStruct((B, H), jnp.float32),
            jax.ShapeDtypeStruct((B, NRP), jnp.float32),
            jax.ShapeDtypeStruct((NRP, H), jnp.float32),
        ],
    )(qw, wif, whf, bf, wib, whb, bb, relp, wrel, brel)


_BLK = 2048


def _proj_body(x, w, b, o):
    o[...] = jnp.dot(x[...], w[...], preferred_element_type=jnp.float32) + b[...]


def _proj(x, w, b):
    n = x.shape[0]
    return pl.pallas_call(
        _proj_body,
        grid=(n // _BLK,),
        in_specs=[
            pl.BlockSpec((_BLK, H), lambda i: (i, 0)),
            pl.BlockSpec((H, H), lambda i: (0, 0)),
            pl.BlockSpec((1, H), lambda i: (0, 0)),
        ],
        out_specs=pl.BlockSpec((_BLK, H), lambda i: (i, 0)),
        out_shape=jax.ShapeDtypeStruct((n, H), jnp.float32),
    )(x, w, b)


def _update_body(a, e, wl, bl, wt, bt, o):
    o[...] = jnp.maximum(
        jnp.dot(a[...], wl[...], preferred_element_type=jnp.float32) + bl[...]
        + jnp.dot(e[...], wt[...], preferred_element_type=jnp.float32) + bt[...],
        0.0)


def _update(a, e, wl, bl, wt, bt):
    n = a.shape[0]
    return pl.pallas_call(
        _update_body,
        grid=(n // _BLK,),
        in_specs=[
            pl.BlockSpec((_BLK, H), lambda i: (i, 0)),
            pl.BlockSpec((_BLK, H), lambda i: (i, 0)),
            pl.BlockSpec((H, H), lambda i: (0, 0)),
            pl.BlockSpec((1, H), lambda i: (0, 0)),
            pl.BlockSpec((H, H), lambda i: (0, 0)),
            pl.BlockSpec((1, H), lambda i: (0, 0)),
        ],
        out_specs=pl.BlockSpec((_BLK, H), lambda i: (i, 0)),
        out_shape=jax.ShapeDtypeStruct((n, H), jnp.float32),
    )(a, e, wl, bl, wt, bt)


def _score_body(e, m, w, b, o):
    s = jnp.dot(e[0], w[...], preferred_element_type=jnp.float32) + b[...]
    s = s + (1.0 - m[0]) * VERY_NEG
    valid = lax.broadcasted_iota(jnp.int32, (M_PAD, 1), 0) < M
    mx = jnp.max(jnp.where(valid, s, -3e38), axis=0, keepdims=True)
    ex = jnp.where(valid, jnp.exp(s - mx), 0.0)
    lse = jnp.log(jnp.sum(ex, axis=0, keepdims=True))
    o[0] = s - mx - lse


def _score(emb, mask, w, b):
    return pl.pallas_call(
        _score_body,
        grid=(B,),
        in_specs=[
            pl.BlockSpec((1, M_PAD, H), lambda i: (i, 0, 0)),
            pl.BlockSpec((1, M_PAD, 1), lambda i: (i, 0, 0)),
            pl.BlockSpec((H, 1), lambda i: (0, 0)),
            pl.BlockSpec((1, 1), lambda i: (0, 0)),
        ],
        out_specs=pl.BlockSpec((1, M_PAD, 1), lambda i: (i, 0, 0)),
        out_shape=jax.ShapeDtypeStruct((B, M_PAD, 1), jnp.float32),
    )(emb, mask, w, b)


def kernel(local_entity, q2e_adj_mat, kb_fact_rel, query_text, answer_dist,
           fact_head, fact_tail, word_emb_table, entity_kge_table,
           rel_emb_table, W_ent, b_ent, W_rel, b_rel, Wi_f, Wh_f, b_f,
           Wi_b, Wh_b, b_b, W_lin, b_lin, W_trans, b_trans, W_score, b_score):
    f32 = jnp.float32
    le = local_entity.astype(jnp.int32)
    ent_idx = jnp.pad(le, ((0, 0), (0, M_PAD - M))).reshape(-1)
    word_idx = jnp.pad(query_text.astype(jnp.int32).T.reshape(-1),
                       (0, 256 - Q * B))
    boff = (jnp.arange(B, dtype=jnp.int32) * M_PAD)[:, None]
    heads = jnp.pad(fact_head.astype(jnp.int32),
                    ((0, 0), (0, F_PAD - F))) + boff
    tails = jnp.pad(fact_tail.astype(jnp.int32), ((0, 0), (0, F_PAD - F)))
    rels = jnp.pad(kb_fact_rel.astype(jnp.int32), ((0, 0), (0, F_PAD - F)),
                   constant_values=NR)
    relp = jnp.pad(rel_emb_table.astype(f32), ((0, NRP - (NR + 1)), (0, 0)))
    zeros_stripe = jnp.zeros((STRIPE, H), f32)
    mask_pad = jnp.pad((le != NE).astype(f32),
                       ((0, 0), (0, M_PAD - M)))[:, :, None]

    kge_rows = _gather_ent(entity_kge_table.astype(f32), ent_idx)
    qw_rows = _gather_word(word_emb_table.astype(f32), word_idx)

    _, score_tab, rel_out = _encode(
        qw_rows[:Q * B], Wi_f, Wh_f, b_f.reshape(1, -1),
        Wi_b, Wh_b, b_b.reshape(1, -1), relp, W_rel, b_rel.reshape(1, -1))

    emb = _proj(kge_rows, W_ent, b_ent.reshape(1, -1))
    for i in range(L):
        agg = _facts(emb, heads, tails, rels, rel_out, score_tab, zeros_stripe)
        emb = _update(agg, emb, W_lin[i], b_lin[i].reshape(1, -1),
                      W_trans[i], b_trans[i].reshape(1, -1))

    out = _score(emb.reshape(B, M_PAD, H), mask_pad, W_score,
                 b_score.reshape(1, 1))
    return out[:, :M, 0]
